# Initial kernel scaffold; baseline (speedup 1.0000x reference)
import jax
import jax.numpy as jnp
from jax.experimental import pallas as pl


def kernel(x, w1f, b1f, w2f, b2f, wd1e, bd1, ws2t, wd1st, cols, spatial_t):
    raise NotImplementedError("write your pallas kernel here")


# speedup vs baseline: 1.5147x; 1.5147x over previous
import jax
import jax.numpy as jnp
from jax.experimental import pallas as pl
from jax.experimental.pallas import tpu as pltpu

_TILE_N = 1024


def _env_kernel(x_ref, w1_ref, b1_ref, w2_ref, b2_ref, wd1e_ref, bd1_ref,
                o_ref):
    h1 = jnp.maximum(
        jnp.dot(x_ref[...], w1_ref[...],
                preferred_element_type=jnp.float32) + b1_ref[...], 0.0)
    h2 = jnp.maximum(
        jnp.dot(h1, w2_ref[...],
                preferred_element_type=jnp.float32) + b2_ref[...], 0.0)
    o_ref[...] = jnp.dot(h2, wd1e_ref[...],
                         preferred_element_type=jnp.float32) + bd1_ref[...]


def _node_kernel(waug_ref, spt_ref, ws2t_ref, cols_ref, wd2t_ref, o_ref):
    S = ws2t_ref.shape[0]
    BH, _ = waug_ref.shape
    B, TN = o_ref.shape
    H3 = BH // B

    cols = cols_ref[...]
    bs1 = cols[0:S, 1:2]
    bs2 = cols[0:S, 2:3]
    ws1t = cols[0:S, 3:6]
    bd2 = cols[0:1, 6:7]

    sp = spt_ref[...]
    s = (bs1
         + ws1t[:, 0:1] * sp[0:1, :]
         + ws1t[:, 1:2] * sp[1:2, :]
         + ws1t[:, 2:3] * sp[2:3, :])
    s = jnp.maximum(s, 0.0)

    sf = jnp.maximum(
        jnp.dot(ws2t_ref[...], s, preferred_element_type=jnp.float32) + bs2,
        0.0)

    sf_aug = jnp.concatenate(
        [sf, jnp.ones((1, TN), jnp.float32)], axis=0)

    pre = jnp.dot(waug_ref[...], sf_aug,
                  preferred_element_type=jnp.float32)

    t = jnp.maximum(pre, 0.0) * wd2t_ref[...]

    o_ref[...] = jnp.sum(t.reshape(B, H3, TN), axis=1) + bd2


def kernel(x, w1f, b1f, w2f, b2f, wd1e, bd1, ws2t, wd1st, cols, spatial_t):
    B = x.shape[0]
    H3 = wd1e.shape[1]
    S = ws2t.shape[0]
    N = spatial_t.shape[1]

    env_proj = pl.pallas_call(
        _env_kernel,
        out_shape=jax.ShapeDtypeStruct((B, H3), jnp.float32),
    )(x, w1f, b1f, w2f, b2f, wd1e, bd1)

    w_aug = jnp.concatenate(
        [jnp.tile(wd1st, (B, 1)), env_proj.reshape(B * H3, 1)], axis=1)
    wd2t = jnp.tile(cols[:, 0:1], (B, 1))

    tile_n = _TILE_N
    n_pad = pl.cdiv(N, tile_n) * tile_n
    if n_pad != N:
        spatial_t = jnp.pad(spatial_t, ((0, 0), (0, n_pad - N)))

    out = pl.pallas_call(
        _node_kernel,
        out_shape=jax.ShapeDtypeStruct((B, n_pad), jnp.float32),
        grid=(n_pad // tile_n,),
        in_specs=[
            pl.BlockSpec((B * H3, S + 1), lambda j: (0, 0)),
            pl.BlockSpec((3, tile_n), lambda j: (0, j)),
            pl.BlockSpec((S, S), lambda j: (0, 0)),
            pl.BlockSpec((H3, 7), lambda j: (0, 0)),
            pl.BlockSpec((B * H3, 1), lambda j: (0, 0)),
        ],
        out_specs=pl.BlockSpec((B, tile_n), lambda j: (0, j)),
        compiler_params=pltpu.CompilerParams(
            dimension_semantics=("parallel",)),
    )(w_aug, spatial_t, ws2t, cols, wd2t)

    if n_pad != N:
        out = out[:, :N]
    return out


# speedup vs baseline: 2.0983x; 1.3852x over previous
import jax
import jax.numpy as jnp
from jax.experimental import pallas as pl
from jax.experimental.pallas import tpu as pltpu

_TILE_N = 4096


def _env_kernel(x_ref, w1_ref, b1_ref, w2_ref, b2_ref, wd1e_ref, bd1_ref,
                o_ref):
    h1 = jnp.maximum(
        jnp.dot(x_ref[...], w1_ref[...],
                preferred_element_type=jnp.float32) + b1_ref[...], 0.0)
    h2 = jnp.maximum(
        jnp.dot(h1, w2_ref[...],
                preferred_element_type=jnp.float32) + b2_ref[...], 0.0)
    o_ref[...] = jnp.dot(h2, wd1e_ref[...],
                         preferred_element_type=jnp.float32) + bd1_ref[...]


def _node_kernel(waug_ref, spt_ref, ws2t_ref, cols_ref, wd2t_ref, o_ref):
    S = ws2t_ref.shape[0]
    BH, _ = waug_ref.shape
    B, TN = o_ref.shape
    H3 = BH // B

    cols = cols_ref[...]
    bs1 = cols[0:S, 1:2]
    bs2 = cols[0:S, 2:3]
    ws1t = cols[0:S, 3:6]
    bd2 = cols[0:1, 6:7]

    sp = spt_ref[...]
    s = (bs1
         + ws1t[:, 0:1] * sp[0:1, :]
         + ws1t[:, 1:2] * sp[1:2, :]
         + ws1t[:, 2:3] * sp[2:3, :])
    s = jnp.maximum(s, 0.0)

    sf = jnp.maximum(
        jnp.dot(ws2t_ref[...], s, preferred_element_type=jnp.float32) + bs2,
        0.0)

    sf_aug = jnp.concatenate(
        [sf, jnp.ones((1, TN), jnp.float32)], axis=0)

    n_chunks = 4
    rows = BH // n_chunks
    acc = None
    for c in range(n_chunks):
        pre = jnp.dot(waug_ref[c * rows:(c + 1) * rows, :], sf_aug,
                      preferred_element_type=jnp.float32)
        t = jnp.maximum(pre, 0.0) * wd2t_ref[c * rows:(c + 1) * rows, :]
        m = rows
        while m > B:
            m //= 2
            t = t[:m, :] + t[m:2 * m, :]
        acc = t if acc is None else acc + t
    o_ref[...] = acc + bd2


def kernel(x, w1f, b1f, w2f, b2f, wd1e, bd1, ws2t, wd1st, cols, spatial_t):
    B = x.shape[0]
    H3 = wd1e.shape[1]
    S = ws2t.shape[0]
    N = spatial_t.shape[1]

    env_proj = pl.pallas_call(
        _env_kernel,
        out_shape=jax.ShapeDtypeStruct((B, H3), jnp.float32),
    )(x, w1f, b1f, w2f, b2f, wd1e, bd1)

    w_aug = jnp.concatenate(
        [jnp.repeat(wd1st, B, axis=0), env_proj.T.reshape(H3 * B, 1)], axis=1)
    wd2t = jnp.repeat(cols[:, 0:1], B, axis=0)

    tile_n = _TILE_N
    n_pad = pl.cdiv(N, tile_n) * tile_n
    if n_pad != N:
        spatial_t = jnp.pad(spatial_t, ((0, 0), (0, n_pad - N)))

    out = pl.pallas_call(
        _node_kernel,
        out_shape=jax.ShapeDtypeStruct((B, n_pad), jnp.float32),
        grid=(n_pad // tile_n,),
        in_specs=[
            pl.BlockSpec((B * H3, S + 1), lambda j: (0, 0)),
            pl.BlockSpec((3, tile_n), lambda j: (0, j)),
            pl.BlockSpec((S, S), lambda j: (0, 0)),
            pl.BlockSpec((H3, 7), lambda j: (0, 0)),
            pl.BlockSpec((B * H3, 1), lambda j: (0, 0)),
        ],
        out_specs=pl.BlockSpec((B, tile_n), lambda j: (0, j)),
        compiler_params=pltpu.CompilerParams(
            dimension_semantics=("parallel",)),
    )(w_aug, spatial_t, ws2t, cols, wd2t)

    if n_pad != N:
        out = out[:, :N]
    return out


# speedup vs baseline: 2.3082x; 1.1001x over previous
import jax
import jax.numpy as jnp
from jax.experimental import pallas as pl
from jax.experimental.pallas import tpu as pltpu

_TILE_N = 4096


def _env_kernel(x_ref, w1_ref, b1_ref, w2_ref, b2_ref, wd1e_ref, bd1_ref,
                o_ref):
    h1 = jnp.maximum(
        jnp.dot(x_ref[...], w1_ref[...],
                preferred_element_type=jnp.float32) + b1_ref[...], 0.0)
    h2 = jnp.maximum(
        jnp.dot(h1, w2_ref[...],
                preferred_element_type=jnp.float32) + b2_ref[...], 0.0)
    o_ref[...] = jnp.dot(h2, wd1e_ref[...],
                         preferred_element_type=jnp.float32) + bd1_ref[...]


def _node_kernel(waug_ref, spt_ref, ws2t_ref, cols_ref, wd2t_ref, o_ref):
    S = ws2t_ref.shape[0]
    BH, _ = waug_ref.shape
    B, TN = o_ref.shape
    H3 = BH // B

    cols = cols_ref[...]
    bs1 = cols[0:S, 1:2]
    bs2 = cols[0:S, 2:3]
    ws1t = cols[0:S, 3:6]
    bd2 = cols[0:1, 6:7]

    sp = spt_ref[...]
    s = (bs1
         + ws1t[:, 0:1] * sp[0:1, :]
         + ws1t[:, 1:2] * sp[1:2, :]
         + ws1t[:, 2:3] * sp[2:3, :])
    s = jnp.maximum(s, 0.0)

    sf = jnp.maximum(
        jnp.dot(ws2t_ref[...], s, preferred_element_type=jnp.float32) + bs2,
        0.0)

    sf_aug = jnp.concatenate(
        [sf, jnp.ones((1, TN), jnp.float32)], axis=0)

    n_chunks = 8
    rows = BH // n_chunks
    acc = None
    for c in range(n_chunks):
        pre = jnp.dot(waug_ref[c * rows:(c + 1) * rows, :], sf_aug,
                      preferred_element_type=jnp.float32)
        t = jnp.maximum(pre, 0.0) * wd2t_ref[c * rows:(c + 1) * rows, :]
        m = rows
        while m > B:
            m //= 2
            t = t[:m, :] + t[m:2 * m, :]
        acc = t if acc is None else acc + t
    o_ref[...] = acc + bd2


def kernel(x, w1f, b1f, w2f, b2f, wd1e, bd1, ws2t, wd1st, cols, spatial_t):
    B = x.shape[0]
    H3 = wd1e.shape[1]
    S = ws2t.shape[0]
    N = spatial_t.shape[1]

    env_proj = pl.pallas_call(
        _env_kernel,
        out_shape=jax.ShapeDtypeStruct((B, H3), jnp.float32),
    )(x, w1f, b1f, w2f, b2f, wd1e, bd1)

    w_aug = jnp.concatenate(
        [jnp.repeat(wd1st, B, axis=0), env_proj.T.reshape(H3 * B, 1)], axis=1)
    wd2t = jnp.repeat(cols[:, 0:1], B, axis=0)

    tile_n = _TILE_N
    out = pl.pallas_call(
        _node_kernel,
        out_shape=jax.ShapeDtypeStruct((B, N), jnp.float32),
        grid=(pl.cdiv(N, tile_n),),
        in_specs=[
            pl.BlockSpec((B * H3, S + 1), lambda j: (0, 0)),
            pl.BlockSpec((3, tile_n), lambda j: (0, j)),
            pl.BlockSpec((S, S), lambda j: (0, 0)),
            pl.BlockSpec((H3, 7), lambda j: (0, 0)),
            pl.BlockSpec((B * H3, 1), lambda j: (0, 0)),
        ],
        out_specs=pl.BlockSpec((B, tile_n), lambda j: (0, j)),
        compiler_params=pltpu.CompilerParams(
            dimension_semantics=("parallel",)),
    )(w_aug, spatial_t, ws2t, cols, wd2t)
    return out


# speedup vs baseline: 2.7658x; 1.1983x over previous
import jax
import jax.numpy as jnp
from jax.experimental import pallas as pl
from jax.experimental.pallas import tpu as pltpu

_TILE_N = 16384
_H_MXU = 48
_MXU_CHUNK = 128


def _env_kernel(x_ref, w1_ref, b1_ref, w2_ref, b2_ref, wd1e_ref, bd1_ref,
                o_ref):
    h1 = jnp.maximum(
        jnp.dot(x_ref[...], w1_ref[...],
                preferred_element_type=jnp.float32) + b1_ref[...], 0.0)
    h2 = jnp.maximum(
        jnp.dot(h1, w2_ref[...],
                preferred_element_type=jnp.float32) + b2_ref[...], 0.0)
    o_ref[...] = jnp.dot(h2, wd1e_ref[...],
                         preferred_element_type=jnp.float32) + bd1_ref[...]


def _node_kernel(waug_ref, spt_ref, ws2t_ref, cols_ref, wd2t_ref, wv_ref,
                 envv_ref, wd2v_ref, o_ref):
    S = ws2t_ref.shape[0]
    BH = waug_ref.shape[0]
    B, TN = o_ref.shape
    HV = wv_ref.shape[0]

    cols = cols_ref[...]
    bs1 = cols[0:S, 1:2]
    bs2 = cols[0:S, 2:3]
    ws1t = cols[0:S, 3:6]
    bd2 = cols[0:1, 6:7]

    sp = spt_ref[...]
    s = (bs1
         + ws1t[:, 0:1] * sp[0:1, :]
         + ws1t[:, 1:2] * sp[1:2, :]
         + ws1t[:, 2:3] * sp[2:3, :])
    s = jnp.maximum(s, 0.0)

    sf = jnp.maximum(
        jnp.dot(ws2t_ref[...], s, preferred_element_type=jnp.float32) + bs2,
        0.0)

    sf_aug = jnp.concatenate(
        [sf, jnp.ones((1, TN), jnp.float32)], axis=0)

    sp_v = jnp.dot(wv_ref[...], sf,
                   preferred_element_type=jnp.float32)

    rows = _MXU_CHUNK
    assert BH % rows == 0, "H_MXU*B must be a multiple of the chunk size"
    wd2w = wd2t_ref[...]
    acc = None
    for c in range(BH // rows):
        pre = jnp.dot(waug_ref[c * rows:(c + 1) * rows, :], sf_aug,
                      preferred_element_type=jnp.float32)
        t = jnp.maximum(pre, 0.0) * wd2w[c * rows:(c + 1) * rows, :]
        m = rows
        while m > B:
            m //= 2
            t = t[:m, :] + t[m:2 * m, :]
        acc = t if acc is None else acc + t

    envv = envv_ref[...]
    wd2v = wd2v_ref[...]
    terms = []
    for i in range(HV):
        th = jnp.maximum(sp_v[i:i + 1, :] + envv[:, i:i + 1], 0.0)
        terms.append(th * wd2v[i:i + 1, 0:1])
    while len(terms) > 1:
        nxt = [terms[i] + terms[i + 1] for i in range(0, len(terms) - 1, 2)]
        if len(terms) % 2:
            nxt.append(terms[-1])
        terms = nxt
    o_ref[...] = acc + terms[0] + bd2


def kernel(x, w1f, b1f, w2f, b2f, wd1e, bd1, ws2t, wd1st, cols, spatial_t):
    B = x.shape[0]
    H3 = wd1e.shape[1]
    S = ws2t.shape[0]
    N = spatial_t.shape[1]
    hm = _H_MXU

    env_proj = pl.pallas_call(
        _env_kernel,
        out_shape=jax.ShapeDtypeStruct((B, H3), jnp.float32),
    )(x, w1f, b1f, w2f, b2f, wd1e, bd1)

    w_aug = jnp.concatenate(
        [jnp.repeat(wd1st[:hm], B, axis=0),
         env_proj[:, :hm].T.reshape(hm * B, 1)], axis=1)
    wd2t = jnp.repeat(cols[:hm, 0:1], B, axis=0)
    wv = wd1st[hm:]
    envv = env_proj[:, hm:]
    wd2v = cols[hm:, 0:1]

    tile_n = _TILE_N
    hv = H3 - hm
    out = pl.pallas_call(
        _node_kernel,
        out_shape=jax.ShapeDtypeStruct((B, N), jnp.float32),
        grid=(pl.cdiv(N, tile_n),),
        in_specs=[
            pl.BlockSpec((hm * B, S + 1), lambda j: (0, 0)),
            pl.BlockSpec((3, tile_n), lambda j: (0, j)),
            pl.BlockSpec((S, S), lambda j: (0, 0)),
            pl.BlockSpec((H3, 7), lambda j: (0, 0)),
            pl.BlockSpec((hm * B, 1), lambda j: (0, 0)),
            pl.BlockSpec((hv, S), lambda j: (0, 0)),
            pl.BlockSpec((B, hv), lambda j: (0, 0)),
            pl.BlockSpec((hv, 1), lambda j: (0, 0)),
        ],
        out_specs=pl.BlockSpec((B, tile_n), lambda j: (0, j)),
        compiler_params=pltpu.CompilerParams(
            dimension_semantics=("parallel",)),
    )(w_aug, spatial_t, ws2t, cols, wd2t, wv, envv, wd2v)
    return out
